# Initial kernel scaffold; baseline (speedup 1.0000x reference)
import jax
import jax.numpy as jnp
from jax.experimental import pallas as pl


def kernel(zt, ztm1):
    raise NotImplementedError("write your pallas kernel here")


# speedup vs baseline: 2.3271x; 2.3271x over previous
import functools
import math

import jax
import jax.numpy as jnp
from jax.experimental import pallas as pl

_Z_DIM = 10


def _affine_consts():
    p = 0.75
    zs = []
    for k in range(_Z_DIM):
        geo = k * math.log(1.0 - p) + math.log(p)
        log_comb = (
            math.lgamma(_Z_DIM + 1.0)
            - math.lgamma(k + 1.0)
            - math.lgamma(_Z_DIM - k + 1.0)
        )
        zs.append(log_comb + geo)
    mx = max(zs)
    z = mx + math.log(sum(math.exp(v - mx) for v in zs))
    a = math.log(1.0 - p)
    b = math.log(p) - z
    return a, b


def _pairwise_kernel(zt_ref, ztm1_t_ref, out_ref, *, a, b):
    acc = jnp.zeros(out_ref.shape, dtype=jnp.float32)
    for d in range(_Z_DIM):
        col = zt_ref[:, d : d + 1]
        row = ztm1_t_ref[d : d + 1, :]
        acc = acc + jnp.abs(col - row)
    k = jnp.floor(acc)
    out_ref[...] = k * a + b


def kernel(zt, ztm1):
    m, zdim = zt.shape
    n = ztm1.shape[0]
    a, b = _affine_consts()
    ztm1_t = ztm1.T

    bi = 512
    grid = (m // bi,)
    return pl.pallas_call(
        functools.partial(_pairwise_kernel, a=a, b=b),
        grid=grid,
        in_specs=[
            pl.BlockSpec((bi, zdim), lambda i: (i, 0)),
            pl.BlockSpec((zdim, n), lambda i: (0, 0)),
        ],
        out_specs=pl.BlockSpec((bi, n), lambda i: (i, 0)),
        out_shape=jax.ShapeDtypeStruct((m, n), jnp.float32),
    )(zt, ztm1_t)
